# baseline (speedup 1.0000x reference)
import jax
import jax.numpy as jnp
import numpy as np
from jax import lax
from jax.experimental import pallas as pl
from jax.experimental.pallas import tpu as pltpu
from jax.experimental.pallas import tpu_sc as plsc

VOCAB = 100000
D = 128
B = 1024
L = 200

_NUM_CORES = 2
_NUM_SUBCORES = 16
_NW = _NUM_CORES * _NUM_SUBCORES
_ROWS = B * L
_ROWS_PER_W = _ROWS // _NW
_CHUNK = 64
_NCHUNK = _ROWS_PER_W // _CHUNK


def _sin_pe(max_len, d_model):
    pos = np.arange(max_len, dtype=np.float32)[:, None]
    div = np.exp(
        np.arange(0, d_model, 2, dtype=np.float32) * -(np.log(10000.0) / d_model)
    )
    pe = np.zeros((max_len, d_model), dtype=np.float32)
    pe[:, 0::2] = np.sin(pos * div)
    pe[:, 1::2] = np.cos(pos * div)
    return pe


_PE = _sin_pe(L, D)


def _embed_kernel(tok_idx_hbm, combo_idx_hbm, tok_table_hbm, combo_hbm, out_hbm,
                  idx_t, idx_c, combo_v, rows0, rows1, out0, out1,
                  gsem0, gsem1, osem0, osem1):
    wid = lax.axis_index("s") * _NUM_CORES + lax.axis_index("c")
    base = wid * _ROWS_PER_W
    rows = (rows0, rows1)
    outs = (out0, out1)
    gsems = (gsem0, gsem1)
    osems = (osem0, osem1)

    pltpu.sync_copy(tok_idx_hbm.at[pl.ds(base, _ROWS_PER_W)], idx_t)
    pltpu.sync_copy(combo_idx_hbm.at[pl.ds(base, _ROWS_PER_W)], idx_c)
    pltpu.sync_copy(combo_hbm, combo_v)

    def gather(g, b):
        return pltpu.async_copy(
            tok_table_hbm.at[idx_t.at[pl.ds(g * _CHUNK, _CHUNK)]],
            rows[b], gsems[b])

    gather(0, 0)

    def pair_body(p, _):
        for bb in range(2):
            g = p * 2 + bb
            pltpu.make_async_copy(
                tok_table_hbm.at[idx_t.at[pl.ds(0, _CHUNK)]],
                rows[bb], gsems[bb]).wait()
            if bb == 0:
                gather(g + 1, 1)
            else:
                @pl.when(p < _NCHUNK // 2 - 1)
                def _():
                    gather(g + 1, 0)
            @pl.when(p >= 1)
            def _():
                pltpu.make_async_copy(
                    outs[bb], out_hbm.at[pl.ds(0, _CHUNK)], osems[bb]).wait()

            def grp_body(q, _):
                r0 = q * 16
                cvec = idx_c[pl.ds(g * _CHUNK + r0, 16)]
                for j in range(16):
                    c = cvec[j]
                    for k in range(D // 16):
                        s = pl.ds(k * 16, 16)
                        outs[bb][r0 + j, s] = rows[bb][r0 + j, s] + combo_v[c, s]
                return ()

            lax.fori_loop(0, _CHUNK // 16, grp_body, ())
            pltpu.async_copy(
                outs[bb], out_hbm.at[pl.ds(base + g * _CHUNK, _CHUNK)], osems[bb])
        return ()

    lax.fori_loop(0, _NCHUNK // 2, pair_body, ())
    for bb in range(2):
        pltpu.make_async_copy(
            outs[bb], out_hbm.at[pl.ds(0, _CHUNK)], osems[bb]).wait()


@jax.jit
def kernel(sequence, segment_label, token_table, seg_table):
    tok_idx = sequence.reshape(-1).astype(jnp.int32)
    pos = jnp.arange(L, dtype=jnp.int32)
    combo_idx = (segment_label.astype(jnp.int32) * L + pos[None, :]).reshape(-1)
    combo = (seg_table[:, None, :] + jnp.asarray(_PE)[None, :, :]).reshape(3 * L, D)

    mesh = plsc.VectorSubcoreMesh(core_axis_name="c", subcore_axis_name="s")
    run = pl.kernel(
        _embed_kernel,
        mesh=mesh,
        out_type=jax.ShapeDtypeStruct((_ROWS, D), jnp.float32),
        scratch_types=[
            pltpu.VMEM((_ROWS_PER_W,), jnp.int32),
            pltpu.VMEM((_ROWS_PER_W,), jnp.int32),
            pltpu.VMEM((3 * L, D), jnp.float32),
            pltpu.VMEM((_CHUNK, D), jnp.float32),
            pltpu.VMEM((_CHUNK, D), jnp.float32),
            pltpu.VMEM((_CHUNK, D), jnp.float32),
            pltpu.VMEM((_CHUNK, D), jnp.float32),
            pltpu.SemaphoreType.DMA,
            pltpu.SemaphoreType.DMA,
            pltpu.SemaphoreType.DMA,
            pltpu.SemaphoreType.DMA,
        ],
    )
    out = run(tok_idx, combo_idx, token_table, combo)
    return out.reshape(B, L, D)
